# baseline (speedup 1.0000x reference)
import functools

import jax
import jax.numpy as jnp
from jax import lax
from jax.experimental import pallas as pl
from jax.experimental.pallas import tpu as pltpu
from jax.experimental.pallas import tpu_sc as plsc

_N = 10000
_E = 160000
_IN = 128
_NBR = 16
_OUT = 128
_H = 2
_F = 2 * _IN + _NBR
_HID = int(_F / 1.5)

_NC = 2
_NS = 16
_NW = _NC * _NS

_C = 2
_EC = _E // _C

_GW = 2 * _EC // _NW
_GCH = 128
_GFULL = _GW // _GCH
_GTAIL = _GW - _GFULL * _GCH

_BE = 2000
_GRID = _EC // _BE
_W1W = 896

_SCH = 128
_EWP = 2560
_SNCH = _EWP // _SCH
_AGG = _NW * _EWP
_NACC = 10240
_NPT = _NACC // _NS


@functools.cache
def _sc_mesh():
    return plsc.VectorSubcoreMesh(
        core_axis_name="c", subcore_axis_name="s",
        num_cores=_NC, num_subcores=_NS)


@functools.cache
def _build_gather():
    @functools.partial(
        pl.kernel,
        out_type=jax.ShapeDtypeStruct((2 * _EC, _IN), jnp.float32),
        mesh=_sc_mesh(),
        scratch_types=[
            pltpu.VMEM((_GW,), jnp.int32),
            pltpu.VMEM((_GCH, _IN), jnp.float32),
            pltpu.VMEM((_GCH, _IN), jnp.float32),
            pltpu.SemaphoreType.DMA,
            pltpu.SemaphoreType.DMA,
            pltpu.SemaphoreType.DMA,
            pltpu.SemaphoreType.DMA,
        ],
    )
    def _gather_rows(x_hbm, idx_hbm, out_hbm, idx_v,
                     bufa, bufb, semga, semgb, semwa, semwb):
        wid = lax.axis_index("s") * _NC + lax.axis_index("c")
        base = wid * _GW
        pltpu.sync_copy(idx_hbm.at[pl.ds(base, _GW)], idx_v)

        def g(c, buf, sem, n=_GCH):
            off = pl.multiple_of(c * _GCH, _GCH)
            pltpu.async_copy(x_hbm.at[idx_v.at[pl.ds(off, n)]],
                             buf.at[pl.ds(0, n)], sem)

        def w(c, buf, sem, n=_GCH):
            off = pl.multiple_of(c * _GCH, _GCH)
            pltpu.async_copy(buf.at[pl.ds(0, n)],
                             out_hbm.at[pl.ds(base + off, n)], sem)

        def wait(buf, sem, n=_GCH):
            pltpu.make_async_copy(
                x_hbm.at[pl.ds(0, n)], buf.at[pl.ds(0, n)], sem).wait()

        g(0, bufa, semga)

        def body(k, carry):
            c0 = 2 * k
            wait(bufa, semga)
            w(c0, bufa, semwa)

            @pl.when(k > 0)
            def _():
                wait(bufb, semwb)

            g(c0 + 1, bufb, semgb)
            wait(bufb, semgb)
            w(c0 + 1, bufb, semwb)
            wait(bufa, semwa)
            g(c0 + 2, bufa, semga)
            return carry

        lax.fori_loop(0, (_GFULL - 1) // 2, body, 0)
        wait(bufa, semga)
        w(_GFULL - 1, bufa, semwa)
        wait(bufb, semwb)
        g(_GFULL, bufb, semgb, _GTAIL)
        wait(bufb, semgb, _GTAIL)
        w(_GFULL, bufb, semwb, _GTAIL)
        wait(bufa, semwa)
        wait(bufb, semwb, _GTAIL)

    return _gather_rows


def _mlp_body(xi_ref, ea_ref, xj_ref, w1_ref, b1_ref,
              w2d_ref, b2d_ref, w2m0_ref, w2m1_ref, b2m_ref, out_ref):
    fea = jnp.concatenate(
        [xi_ref[...].astype(jnp.bfloat16),
         xj_ref[...].astype(jnp.bfloat16),
         ea_ref[...].astype(jnp.bfloat16)], axis=1)
    h = jnp.dot(fea, w1_ref[...], preferred_element_type=jnp.float32)
    hb = h.astype(jnp.bfloat16) + b1_ref[...]
    hb = jnp.where(hb >= 0, hb, jnp.bfloat16(0.01) * hb)
    d = jnp.dot(hb, w2d_ref[...], preferred_element_type=jnp.float32)
    d += b2d_ref[...]
    w0 = 1.0 / (1.0 + jnp.exp(-d))
    m0 = jnp.dot(hb[:, 0:256], w2m0_ref[...],
                 preferred_element_type=jnp.float32) + b2m_ref[0:1, :]
    m1 = jnp.dot(hb[:, 256:512], w2m1_ref[...],
                 preferred_element_type=jnp.float32) + b2m_ref[1:2, :]
    out_ref[...] = 0.5 * (m1 + w0 * (m0 - m1))


def _edge_mlp(xij, edge_attr, w1, b1, w2d, b2d, w2m0, w2m1, b2m):
    const = lambda i: (0, 0)
    return pl.pallas_call(
        _mlp_body,
        grid=(_GRID,),
        in_specs=[
            pl.BlockSpec((_BE, _IN), lambda i: (i, 0)),
            pl.BlockSpec((_BE, _NBR), lambda i: (i, 0)),
            pl.BlockSpec((_BE, _IN), lambda i: (i + _GRID, 0)),
            pl.BlockSpec((_F, _W1W), const),
            pl.BlockSpec((1, _W1W), const),
            pl.BlockSpec((_W1W, 1), const),
            pl.BlockSpec((1, 1), const),
            pl.BlockSpec((256, _OUT), const),
            pl.BlockSpec((256, _OUT), const),
            pl.BlockSpec((_H, _OUT), const),
        ],
        out_specs=pl.BlockSpec((_BE, _OUT), lambda i: (i, 0)),
        out_shape=jax.ShapeDtypeStruct((_AGG, _OUT), jnp.float32),
    )(xij, edge_attr, xij, w1, b1, w2d, b2d, w2m0, w2m1, b2m)


@functools.cache
def _build_scatter():
    @functools.partial(
        pl.kernel,
        out_type=jax.ShapeDtypeStruct((_NC, _NACC, _OUT), jnp.float32),
        mesh=_sc_mesh(),
        scratch_types=[
            pltpu.VMEM((_SNCH, _SCH), jnp.int32),
            pltpu.VMEM((_SCH, _OUT), jnp.float32),
            pltpu.VMEM((_SCH, _OUT), jnp.float32),
            pltpu.VMEM_SHARED((_NACC, _OUT), jnp.float32),
            pltpu.SemaphoreType.DMA,
            pltpu.SemaphoreType.DMA,
        ],
    )
    def _scatter_add(aggr_hbm, dstr_hbm, zeros_hbm, out_hbm,
                     idx_v, bufa, bufb, acc, semra, semrb):
        cid = lax.axis_index("c")
        sid = lax.axis_index("s")
        wid = sid * _NC + cid
        pltpu.sync_copy(zeros_hbm, acc.at[pl.ds(sid * _NPT, _NPT)])
        pltpu.sync_copy(dstr_hbm.at[wid], idx_v)
        plsc.subcore_barrier()

        def rd(j, buf, sem):
            off = pl.multiple_of(j * _SCH, _SCH)
            pltpu.async_copy(
                aggr_hbm.at[pl.ds(wid * _EWP + off, _SCH)], buf, sem)

        def wait(buf, sem):
            pltpu.make_async_copy(
                aggr_hbm.at[pl.ds(0, _SCH)], buf, sem).wait()

        rd(0, bufa, semra)

        def body(k, carry):
            j0 = 2 * k
            wait(bufa, semra)
            rd(j0 + 1, bufb, semrb)
            pltpu.sync_copy(bufa, acc.at[idx_v.at[j0]], add=True)
            wait(bufb, semrb)

            @pl.when(k + 1 < _SNCH // 2)
            def _():
                rd(j0 + 2, bufa, semra)

            pltpu.sync_copy(bufb, acc.at[idx_v.at[j0 + 1]], add=True)
            return carry

        lax.fori_loop(0, _SNCH // 2, body, 0)
        plsc.subcore_barrier()
        pltpu.sync_copy(acc.at[pl.ds(sid * _NPT, _NPT)],
                        out_hbm.at[cid, pl.ds(sid * _NPT, _NPT)])

    return _scatter_add


def _add_body(a_ref, b_ref, c_ref, d_ref, o_ref):
    o_ref[...] = (a_ref[0] + b_ref[0]) + (c_ref[0] + d_ref[0])


def _add_partials(pa, pb):
    bn = 2000
    spec0 = pl.BlockSpec((1, bn, _OUT), lambda i: (0, i, 0))
    spec1 = pl.BlockSpec((1, bn, _OUT), lambda i: (1, i, 0))
    return pl.pallas_call(
        _add_body,
        grid=(_N // bn,),
        in_specs=[spec0, spec1, spec0, spec1],
        out_specs=pl.BlockSpec((bn, _OUT), lambda i: (i, 0)),
        out_shape=jax.ShapeDtypeStruct((_N, _OUT), jnp.float32),
    )(pa, pa, pb, pb)


def kernel(x, edge_index, edge_attr, W1a, b1a, W2a, b2a, W1m, b1m, W2m, b2m):
    def chunk(w, b):
        wt = jnp.pad(w.T, ((0, 0), (0, 256 - _HID)))
        bt = jnp.pad(b, (0, 256 - _HID))
        return wt, bt

    m0w, m0b = chunk(W1m[0], b1m[0])
    m1w, m1b = chunk(W1m[1], b1m[1])
    aw = jnp.pad(jnp.concatenate([W1a[0].T, W1a[1].T], axis=1),
                 ((0, 0), (0, _W1W - 512 - 2 * _HID)))
    ab = jnp.pad(jnp.concatenate([b1a[0], b1a[1]]), (0, _W1W - 512 - 2 * _HID))
    w1 = jnp.concatenate([m0w, m1w, aw], axis=1)
    b1 = jnp.concatenate([m0b, m1b, ab])[None, :]
    w1r = jnp.concatenate(
        [w1[:_IN], w1[_IN + _NBR:], w1[_IN:_IN + _NBR]],
        axis=0).astype(jnp.bfloat16)
    b1b = b1.astype(jnp.bfloat16)
    w2a = jnp.zeros((_W1W, _H), jnp.float32)
    w2a = w2a.at[512:512 + _HID, 0].set(W2a[0, 0])
    w2a = w2a.at[512 + _HID:512 + 2 * _HID, 1].set(W2a[1, 0])
    w2d = (w2a[:, 0:1] - w2a[:, 1:2]).astype(jnp.bfloat16)
    b2d = (b2a[0] - b2a[1]).reshape(1, 1)
    w2m0 = jnp.pad(W2m[0].T, ((0, 256 - _HID), (0, 0))).astype(jnp.bfloat16)
    w2m1 = jnp.pad(W2m[1].T, ((0, 256 - _HID), (0, 0))).astype(jnp.bfloat16)

    src = edge_index[0]
    dst = edge_index[1]
    idx = jnp.concatenate(
        [src.reshape(_C, _EC), dst.reshape(_C, _EC)], axis=1)
    dstr = jnp.pad(dst.reshape(_C, _EC), ((0, 0), (0, _AGG - _EC)),
                   constant_values=_N).reshape(_C, _NW, _SNCH, _SCH)
    zeros = jnp.zeros((_NPT, _OUT), jnp.float32)

    gather = _build_gather()
    scatter = _build_scatter()
    parts = []
    for c in range(_C):
        xij = gather(x, idx[c])
        aggr = _edge_mlp(xij, edge_attr[c * _EC:(c + 1) * _EC], w1r, b1b,
                         w2d, b2d, w2m0, w2m1, b2m)
        parts.append(scatter(aggr, dstr[c], zeros))
    return _add_partials(*parts)


# speedup vs baseline: 1.0075x; 1.0075x over previous
import functools

import jax
import jax.numpy as jnp
from jax import lax
from jax.experimental import pallas as pl
from jax.experimental.pallas import tpu as pltpu
from jax.experimental.pallas import tpu_sc as plsc

_N = 10000
_E = 160000
_IN = 128
_NBR = 16
_OUT = 128
_H = 2
_F = 2 * _IN + _NBR
_HID = int(_F / 1.5)

_NC = 2
_NS = 16
_NW = _NC * _NS

_C = 2
_EC = _E // _C

_GW = 2 * _EC // _NW
_GCH = 128
_GFULL = _GW // _GCH
_GTAIL = _GW - _GFULL * _GCH

_BE = 1600
_GRID = _EC // _BE
_W1W = 896

_SCH = 128
_EWP = 2560
_SNCH = _EWP // _SCH
_AGG = _NW * _EWP
_NACC = 10240
_NPT = _NACC // _NS


@functools.cache
def _sc_mesh():
    return plsc.VectorSubcoreMesh(
        core_axis_name="c", subcore_axis_name="s",
        num_cores=_NC, num_subcores=_NS)


@functools.cache
def _build_gather():
    @functools.partial(
        pl.kernel,
        out_type=jax.ShapeDtypeStruct((2 * _EC, _IN), jnp.float32),
        mesh=_sc_mesh(),
        scratch_types=[
            pltpu.VMEM((_GW,), jnp.int32),
            pltpu.VMEM((_GCH, _IN), jnp.float32),
            pltpu.VMEM((_GCH, _IN), jnp.float32),
            pltpu.SemaphoreType.DMA,
            pltpu.SemaphoreType.DMA,
            pltpu.SemaphoreType.DMA,
            pltpu.SemaphoreType.DMA,
        ],
    )
    def _gather_rows(x_hbm, idx_hbm, out_hbm, idx_v,
                     bufa, bufb, semga, semgb, semwa, semwb):
        wid = lax.axis_index("s") * _NC + lax.axis_index("c")
        base = wid * _GW
        pltpu.sync_copy(idx_hbm.at[pl.ds(base, _GW)], idx_v)

        def g(c, buf, sem, n=_GCH):
            off = pl.multiple_of(c * _GCH, _GCH)
            pltpu.async_copy(x_hbm.at[idx_v.at[pl.ds(off, n)]],
                             buf.at[pl.ds(0, n)], sem)

        def w(c, buf, sem, n=_GCH):
            off = pl.multiple_of(c * _GCH, _GCH)
            pltpu.async_copy(buf.at[pl.ds(0, n)],
                             out_hbm.at[pl.ds(base + off, n)], sem)

        def wait(buf, sem, n=_GCH):
            pltpu.make_async_copy(
                x_hbm.at[pl.ds(0, n)], buf.at[pl.ds(0, n)], sem).wait()

        g(0, bufa, semga)

        def body(k, carry):
            c0 = 2 * k
            wait(bufa, semga)
            w(c0, bufa, semwa)

            @pl.when(k > 0)
            def _():
                wait(bufb, semwb)

            g(c0 + 1, bufb, semgb)
            wait(bufb, semgb)
            w(c0 + 1, bufb, semwb)
            wait(bufa, semwa)
            g(c0 + 2, bufa, semga)
            return carry

        lax.fori_loop(0, (_GFULL - 1) // 2, body, 0)
        wait(bufa, semga)
        w(_GFULL - 1, bufa, semwa)
        wait(bufb, semwb)
        g(_GFULL, bufb, semgb, _GTAIL)
        wait(bufb, semgb, _GTAIL)
        w(_GFULL, bufb, semwb, _GTAIL)
        wait(bufa, semwa)
        wait(bufb, semwb, _GTAIL)

    return _gather_rows


def _mlp_body(xi_ref, ea_ref, xj_ref, w1_ref, b1_ref,
              w2d_ref, b2d_ref, w2m0_ref, w2m1_ref, b2m_ref, out_ref):
    fea = jnp.concatenate(
        [xi_ref[...].astype(jnp.bfloat16),
         xj_ref[...].astype(jnp.bfloat16),
         ea_ref[...].astype(jnp.bfloat16)], axis=1)
    h = jnp.dot(fea, w1_ref[...], preferred_element_type=jnp.float32)
    hb = h.astype(jnp.bfloat16) + b1_ref[...]
    hb = jnp.where(hb >= 0, hb, jnp.bfloat16(0.01) * hb)
    d = jnp.dot(hb, w2d_ref[...], preferred_element_type=jnp.float32)
    d += b2d_ref[...]
    w0 = 1.0 / (1.0 + jnp.exp(-d))
    m0 = jnp.dot(hb[:, 0:256], w2m0_ref[...],
                 preferred_element_type=jnp.float32) + b2m_ref[0:1, :]
    m1 = jnp.dot(hb[:, 256:512], w2m1_ref[...],
                 preferred_element_type=jnp.float32) + b2m_ref[1:2, :]
    out_ref[...] = 0.5 * (m1 + w0 * (m0 - m1))


def _edge_mlp(xij, edge_attr, w1, b1, w2d, b2d, w2m0, w2m1, b2m):
    const = lambda i: (0, 0)
    return pl.pallas_call(
        _mlp_body,
        grid=(_GRID,),
        in_specs=[
            pl.BlockSpec((_BE, _IN), lambda i: (i, 0)),
            pl.BlockSpec((_BE, _NBR), lambda i: (i, 0)),
            pl.BlockSpec((_BE, _IN), lambda i: (i + _GRID, 0)),
            pl.BlockSpec((_F, _W1W), const),
            pl.BlockSpec((1, _W1W), const),
            pl.BlockSpec((_W1W, 1), const),
            pl.BlockSpec((1, 1), const),
            pl.BlockSpec((256, _OUT), const),
            pl.BlockSpec((256, _OUT), const),
            pl.BlockSpec((_H, _OUT), const),
        ],
        out_specs=pl.BlockSpec((_BE, _OUT), lambda i: (i, 0)),
        out_shape=jax.ShapeDtypeStruct((_AGG, _OUT), jnp.float32),
    )(xij, edge_attr, xij, w1, b1, w2d, b2d, w2m0, w2m1, b2m)


@functools.cache
def _build_scatter():
    @functools.partial(
        pl.kernel,
        out_type=jax.ShapeDtypeStruct((_NC, _NACC, _OUT), jnp.float32),
        mesh=_sc_mesh(),
        scratch_types=[
            pltpu.VMEM((_SNCH, _SCH), jnp.int32),
            pltpu.VMEM((_SCH, _OUT), jnp.float32),
            pltpu.VMEM((_SCH, _OUT), jnp.float32),
            pltpu.VMEM_SHARED((_NACC, _OUT), jnp.float32),
            pltpu.SemaphoreType.DMA,
            pltpu.SemaphoreType.DMA,
        ],
    )
    def _scatter_add(aggr_hbm, dstr_hbm, zeros_hbm, out_hbm,
                     idx_v, bufa, bufb, acc, semra, semrb):
        cid = lax.axis_index("c")
        sid = lax.axis_index("s")
        wid = sid * _NC + cid
        pltpu.sync_copy(zeros_hbm, acc.at[pl.ds(sid * _NPT, _NPT)])
        pltpu.sync_copy(dstr_hbm.at[wid], idx_v)
        plsc.subcore_barrier()

        def rd(j, buf, sem):
            off = pl.multiple_of(j * _SCH, _SCH)
            pltpu.async_copy(
                aggr_hbm.at[pl.ds(wid * _EWP + off, _SCH)], buf, sem)

        def wait(buf, sem):
            pltpu.make_async_copy(
                aggr_hbm.at[pl.ds(0, _SCH)], buf, sem).wait()

        rd(0, bufa, semra)

        def body(k, carry):
            j0 = 2 * k
            wait(bufa, semra)
            rd(j0 + 1, bufb, semrb)
            pltpu.sync_copy(bufa, acc.at[idx_v.at[j0]], add=True)
            wait(bufb, semrb)

            @pl.when(k + 1 < _SNCH // 2)
            def _():
                rd(j0 + 2, bufa, semra)

            pltpu.sync_copy(bufb, acc.at[idx_v.at[j0 + 1]], add=True)
            return carry

        lax.fori_loop(0, _SNCH // 2, body, 0)
        plsc.subcore_barrier()
        pltpu.sync_copy(acc.at[pl.ds(sid * _NPT, _NPT)],
                        out_hbm.at[cid, pl.ds(sid * _NPT, _NPT)])

    return _scatter_add


def _add_body(a_ref, b_ref, c_ref, d_ref, o_ref):
    o_ref[...] = (a_ref[0] + b_ref[0]) + (c_ref[0] + d_ref[0])


def _add_partials(pa, pb):
    bn = 2000
    spec0 = pl.BlockSpec((1, bn, _OUT), lambda i: (0, i, 0))
    spec1 = pl.BlockSpec((1, bn, _OUT), lambda i: (1, i, 0))
    return pl.pallas_call(
        _add_body,
        grid=(_N // bn,),
        in_specs=[spec0, spec1, spec0, spec1],
        out_specs=pl.BlockSpec((bn, _OUT), lambda i: (i, 0)),
        out_shape=jax.ShapeDtypeStruct((_N, _OUT), jnp.float32),
    )(pa, pa, pb, pb)


def kernel(x, edge_index, edge_attr, W1a, b1a, W2a, b2a, W1m, b1m, W2m, b2m):
    def chunk(w, b):
        wt = jnp.pad(w.T, ((0, 0), (0, 256 - _HID)))
        bt = jnp.pad(b, (0, 256 - _HID))
        return wt, bt

    m0w, m0b = chunk(W1m[0], b1m[0])
    m1w, m1b = chunk(W1m[1], b1m[1])
    aw = jnp.pad(jnp.concatenate([W1a[0].T, W1a[1].T], axis=1),
                 ((0, 0), (0, _W1W - 512 - 2 * _HID)))
    ab = jnp.pad(jnp.concatenate([b1a[0], b1a[1]]), (0, _W1W - 512 - 2 * _HID))
    w1 = jnp.concatenate([m0w, m1w, aw], axis=1)
    b1 = jnp.concatenate([m0b, m1b, ab])[None, :]
    w1r = jnp.concatenate(
        [w1[:_IN], w1[_IN + _NBR:], w1[_IN:_IN + _NBR]],
        axis=0).astype(jnp.bfloat16)
    b1b = b1.astype(jnp.bfloat16)
    w2a = jnp.zeros((_W1W, _H), jnp.float32)
    w2a = w2a.at[512:512 + _HID, 0].set(W2a[0, 0])
    w2a = w2a.at[512 + _HID:512 + 2 * _HID, 1].set(W2a[1, 0])
    w2d = (w2a[:, 0:1] - w2a[:, 1:2]).astype(jnp.bfloat16)
    b2d = (b2a[0] - b2a[1]).reshape(1, 1)
    w2m0 = jnp.pad(W2m[0].T, ((0, 256 - _HID), (0, 0))).astype(jnp.bfloat16)
    w2m1 = jnp.pad(W2m[1].T, ((0, 256 - _HID), (0, 0))).astype(jnp.bfloat16)

    src = edge_index[0]
    dst = edge_index[1]
    idx = jnp.concatenate(
        [src.reshape(_C, _EC), dst.reshape(_C, _EC)], axis=1)
    dstr = jnp.pad(dst.reshape(_C, _EC), ((0, 0), (0, _AGG - _EC)),
                   constant_values=_N).reshape(_C, _NW, _SNCH, _SCH)
    zeros = jnp.zeros((_NPT, _OUT), jnp.float32)

    gather = _build_gather()
    scatter = _build_scatter()
    parts = []
    for c in range(_C):
        xij = gather(x, idx[c])
        aggr = _edge_mlp(xij, edge_attr[c * _EC:(c + 1) * _EC], w1r, b1b,
                         w2d, b2d, w2m0, w2m1, b2m)
        parts.append(scatter(aggr, dstr[c], zeros))
    return _add_partials(*parts)
